# baseline (speedup 1.0000x reference)
import functools

import jax
import jax.numpy as jnp
from jax import lax
from jax.experimental import pallas as pl
from jax.experimental.pallas import tpu as pltpu
from jax.experimental.pallas import tpu_sc as plsc

N_ROWS = 10000
D = 128
DW = D // 2
E_TOTAL = 320000
NC, NS = 2, 16
NW = NC * NS
EDGES_PER_TILE = E_TOTAL // NW
CHUNK = 80
EPT_PAD = 10240
NCHUNK = EPT_PAD // CHUNK
N_PAD = 10240
ROWS_PER_TILE = N_PAD // NS
TRASH_ROW = N_PAD - 1
ZROWS = 16
_HIMASK = -65536


def _make_propagate():
    mesh = plsc.VectorSubcoreMesh(core_axis_name="c", subcore_axis_name="s")

    def body(table_hbm, gidx_hbm, sidx_hbm, out_hbm, gflat, sring, praw,
             rows, zbuf, acc, *sems):
        isem = sems[0:4]
        gsem = sems[4:6]
        ssem = sems[6:8]
        cid = lax.axis_index("c")
        sid = lax.axis_index("s")
        wid = cid * NS + sid

        pltpu.async_copy(gidx_hbm.at[wid], gflat, gsem[0])

        for zi in range(ZROWS):
            for j in range(D // 16):
                zbuf[zi, pl.ds(j * 16, 16)] = jnp.zeros((16,), jnp.float32)
        row0 = sid * ROWS_PER_TILE
        for r in range(ROWS_PER_TILE // ZROWS):
            pltpu.sync_copy(zbuf, acc.at[pl.ds(row0 + r * ZROWS, ZROWS)])
        pltpu.make_async_copy(gidx_hbm.at[wid], gflat, gsem[0]).wait()
        plsc.subcore_barrier()

        def i_start(i, q):
            pltpu.async_copy(sidx_hbm.at[wid, i], sring.at[q], isem[q])

        def i_wait(i, q):
            pltpu.make_async_copy(sidx_hbm.at[wid, i], sring.at[q],
                                  isem[q]).wait()

        def g_start(i, b):
            pltpu.async_copy(table_hbm.at[gflat.at[pl.ds(i * CHUNK, CHUNK)]],
                             praw.at[b], gsem[b])

        def g_wait(i, b):
            pltpu.make_async_copy(table_hbm.at[pl.ds(0, CHUNK)],
                                  praw.at[b], gsem[b]).wait()

        def unpack(b):
            mask = jnp.full((16,), _HIMASK, jnp.int32)

            def urow(r4, _):
                for dr in range(4):
                    r = r4 * 4 + dr
                    for j in range(DW // 16):
                        w = praw[b, r, pl.ds(j * 16, 16)]
                        lo = plsc.bitcast(lax.shift_left(w, 16), jnp.float32)
                        hi = plsc.bitcast(lax.bitwise_and(w, mask),
                                          jnp.float32)
                        rows[b, r, pl.ds(j * 16, 16)] = lo
                        rows[b, r, pl.ds(DW + j * 16, 16)] = hi
                return 0
            lax.fori_loop(0, CHUNK // 4, urow, 0)

        def s_start(i, b, q):
            pltpu.async_copy(rows.at[b], acc.at[sring.at[q]], ssem[b],
                             add=True)

        def s_wait(i, b, q):
            pltpu.make_async_copy(rows.at[b], acc.at[pl.ds(0, CHUNK)],
                                  ssem[b]).wait()

        def step(i, b, q, fire):
            s_wait(i - 2, b, (q + 2) % 4)
            if fire:
                i_start(i + 2, (q + 2) % 4)
            g_start(i, b)
            g_wait(i - 1, 1 - b)
            unpack(1 - b)
            i_wait(i - 1, (q + 3) % 4)
            s_start(i - 1, 1 - b, (q + 3) % 4)

        i_start(0, 0)
        i_start(1, 1)
        g_start(0, 0)
        i_start(2, 2)
        g_start(1, 1)
        i_start(3, 3)
        g_wait(0, 0)
        unpack(0)
        i_wait(0, 0)
        s_start(0, 0, 0)
        nfour = (NCHUNK - 4) // 4

        def four(t, _):
            i0 = 2 + 4 * t
            for u in range(4):
                step(i0 + u, u % 2, (2 + u) % 4, True)
            return 0
        lax.fori_loop(0, nfour, four, 0)
        step(NCHUNK - 2, 0, 2, False)
        step(NCHUNK - 1, 1, 3, False)
        s_wait(NCHUNK - 2, 0, 2)
        g_wait(NCHUNK - 1, 1)
        unpack(1)
        i_wait(NCHUNK - 1, 3)
        s_start(NCHUNK - 1, 1, 3)
        s_wait(NCHUNK - 1, 1, 3)
        plsc.subcore_barrier()

        pltpu.sync_copy(acc.at[pl.ds(row0, ROWS_PER_TILE)],
                        out_hbm.at[cid, pl.ds(row0, ROWS_PER_TILE)])

    return pl.kernel(
        body,
        out_type=jax.ShapeDtypeStruct((NC, N_PAD, D), jnp.float32),
        compiler_params=pltpu.CompilerParams(use_tc_tiling_on_sc=False,
                                             needs_layout_passes=False),
        mesh=mesh,
        scratch_types=[
            pltpu.VMEM((EPT_PAD,), jnp.int32),
            pltpu.VMEM((4, CHUNK), jnp.int32),
            pltpu.VMEM((2, CHUNK, DW), jnp.int32),
            pltpu.VMEM((2, CHUNK, D), jnp.float32),
            pltpu.VMEM((ZROWS, D), jnp.float32),
            pltpu.VMEM_SHARED((N_PAD, D), jnp.float32),
        ] + [pltpu.SemaphoreType.DMA] * 8,
    )


_propagate = _make_propagate()


_BLK = 2000
_DOT = functools.partial(
    lax.dot_general,
    dimension_numbers=(((1,), (0,)), ((), ())),
    preferred_element_type=jnp.float32,
    precision=lax.Precision.HIGHEST,
)


def _pack_words(y):
    lo = lax.bitcast_convert_type(y[:, :DW].astype(jnp.bfloat16),
                                  jnp.uint16).astype(jnp.int32)
    hi = lax.bitcast_convert_type(y[:, DW:].astype(jnp.bfloat16),
                                  jnp.uint16).astype(jnp.int32)
    return jnp.bitwise_or(lo, lax.shift_left(hi, 16))


def _first_body(x_ref, so_ref, w_ref, bias_ref, o_ref):
    y = _DOT(x_ref[...], w_ref[...]) + bias_ref[...]
    o_ref[...] = _pack_words(y * so_ref[...])


def _mid_body(a_ref, b_ref, si_ref, so_ref, w_ref, bias_ref, o_ref):
    t = (a_ref[...] + b_ref[...]) * si_ref[...]
    t = jnp.maximum(t, 0.0)
    y = _DOT(t, w_ref[...]) + bias_ref[...]
    o_ref[...] = _pack_words(y * so_ref[...])


def _last_body(a_ref, b_ref, si_ref, o_ref):
    o_ref[...] = (a_ref[...] + b_ref[...]) * si_ref[...]


_ROWB = pl.BlockSpec((_BLK, D), lambda i: (i, 0))
_PKB = pl.BlockSpec((_BLK, DW), lambda i: (i, 0))
_COLB = pl.BlockSpec((_BLK, 1), lambda i: (i, 0))
_WB = pl.BlockSpec((D, D), lambda i: (0, 0))
_BB = pl.BlockSpec((1, D), lambda i: (0, 0))
_GRID = (N_ROWS // _BLK,)
_OSHAPE = jax.ShapeDtypeStruct((N_ROWS, D), jnp.float32)
_PSHAPE = jax.ShapeDtypeStruct((N_ROWS, DW), jnp.int32)

_dense_first = pl.pallas_call(
    _first_body, grid=_GRID, out_shape=_PSHAPE,
    in_specs=[_ROWB, _COLB, _WB, _BB], out_specs=_PKB)

_dense_mid = pl.pallas_call(
    _mid_body, grid=_GRID, out_shape=_PSHAPE,
    in_specs=[_ROWB, _ROWB, _COLB, _COLB, _WB, _BB], out_specs=_PKB)

_dense_last = pl.pallas_call(
    _last_body, grid=_GRID, out_shape=_OSHAPE,
    in_specs=[_ROWB, _ROWB, _COLB], out_specs=_ROWB)


def _pad_idx(idx, fill):
    per_tile = idx.reshape(NW, EDGES_PER_TILE)
    return jnp.pad(per_tile, ((0, 0), (0, EPT_PAD - EDGES_PER_TILE)),
                   constant_values=fill)


def kernel(x, edge_index, D_v_beta, D_e_beta_inv, D_e_alpha, D_v_alpha_inv,
           W1v, b1v, W1e, b1e, W2v, b2v, W2e, b2e):
    src = edge_index[0]
    dst = edge_index[1]
    src_g = _pad_idx(src, 0)
    dst_g = _pad_idx(dst, 0)
    src_s = _pad_idx(src, TRASH_ROW).reshape(NW, NCHUNK, CHUNK)
    dst_s = _pad_idx(dst, TRASH_ROW).reshape(NW, NCHUNK, CHUNK)

    dvb = D_v_beta.reshape(N_ROWS, 1)
    debi = D_e_beta_inv.reshape(N_ROWS, 1)
    dea = D_e_alpha.reshape(N_ROWS, 1)
    dvai = D_v_alpha_inv.reshape(N_ROWS, 1)
    b1v_ = b1v.reshape(1, D)
    b1e_ = b1e.reshape(1, D)
    b2v_ = b2v.reshape(1, D)
    b2e_ = b2e.reshape(1, D)

    h1 = _dense_first(x, dvb, W1v, b1v_)
    p = _propagate(h1, src_g, dst_s)
    e2 = _dense_mid(p[0, :N_ROWS], p[1, :N_ROWS], debi, dea, W1e, b1e_)
    q = _propagate(e2, dst_g, src_s)
    h2 = _dense_mid(q[0, :N_ROWS], q[1, :N_ROWS], dvai, dvb, W2v, b2v_)
    p2 = _propagate(h2, src_g, dst_s)
    e2b = _dense_mid(p2[0, :N_ROWS], p2[1, :N_ROWS], debi, dea, W2e, b2e_)
    q2 = _propagate(e2b, dst_g, src_s)
    return _dense_last(q2[0, :N_ROWS], q2[1, :N_ROWS], dvai)


# speedup vs baseline: 1.4584x; 1.4584x over previous
import functools

import jax
import jax.numpy as jnp
from jax import lax
from jax.experimental import pallas as pl
from jax.experimental.pallas import tpu as pltpu
from jax.experimental.pallas import tpu_sc as plsc

N_ROWS = 10000
D = 128
DW = D // 2
E_TOTAL = 320000
NC, NS = 2, 16
NW = NC * NS
EDGES_PER_TILE = E_TOTAL // NW
CHUNK = 64
EPT_PAD = 10240
NCHUNK = EPT_PAD // CHUNK
N_PAD = 10240
ROWS_PER_TILE = N_PAD // NS
TRASH_ROW = N_PAD - 1
ZROWS = 16
_HIMASK = -65536


def _make_propagate():
    mesh = plsc.VectorSubcoreMesh(core_axis_name="c", subcore_axis_name="s")

    def body(table_hbm, gidx_hbm, sidx_hbm, out_hbm, gflat, sring, praw,
             rows, zbuf, acc, *sems):
        isem = sems[0:4]
        gsem = sems[4:8]
        ssem = sems[8:10]
        cid = lax.axis_index("c")
        sid = lax.axis_index("s")
        wid = cid * NS + sid

        pltpu.async_copy(gidx_hbm.at[wid], gflat, gsem[0])

        for zi in range(ZROWS):
            for j in range(D // 16):
                zbuf[zi, pl.ds(j * 16, 16)] = jnp.zeros((16,), jnp.float32)
        row0 = sid * ROWS_PER_TILE
        for r in range(ROWS_PER_TILE // ZROWS):
            pltpu.sync_copy(zbuf, acc.at[pl.ds(row0 + r * ZROWS, ZROWS)])
        pltpu.make_async_copy(gidx_hbm.at[wid], gflat, gsem[0]).wait()
        plsc.subcore_barrier()

        def i_start(i, p):
            pltpu.async_copy(sidx_hbm.at[wid, i], sring.at[p], isem[p])

        def i_wait(i, p):
            pltpu.make_async_copy(sidx_hbm.at[wid, i], sring.at[p],
                                  isem[p]).wait()

        def g_start(i, p):
            pltpu.async_copy(table_hbm.at[gflat.at[pl.ds(i * CHUNK, CHUNK)]],
                             praw.at[p], gsem[p])

        def g_wait(i, p):
            pltpu.make_async_copy(table_hbm.at[pl.ds(0, CHUNK)],
                                  praw.at[p], gsem[p]).wait()

        def unpack(p, b):
            mask = jnp.full((16,), _HIMASK, jnp.int32)

            @plsc.parallel_loop(0, CHUNK, 1, unroll=4)
            def urow(r):
                for j in range(DW // 16):
                    w = praw[p, r, pl.ds(j * 16, 16)]
                    rows[b, r, pl.ds(j * 16, 16)] = plsc.bitcast(
                        lax.shift_left(w, 16), jnp.float32)
                    rows[b, r, pl.ds(DW + j * 16, 16)] = plsc.bitcast(
                        lax.bitwise_and(w, mask), jnp.float32)

        def s_start(i, b, p):
            pltpu.async_copy(rows.at[b], acc.at[sring.at[p]], ssem[b],
                             add=True)

        def s_wait(i, b):
            pltpu.make_async_copy(rows.at[b], acc.at[pl.ds(0, CHUNK)],
                                  ssem[b]).wait()

        def step(i, b, q, fire_i, fire_g):
            s_wait(i - 2, b)
            if fire_i:
                i_start(i + 2, (q + 2) % 4)
            g_wait(i - 1, (q + 3) % 4)
            unpack((q + 3) % 4, 1 - b)
            if fire_g:
                g_start(i + 3, (q + 3) % 4)
            i_wait(i - 1, (q + 3) % 4)
            s_start(i - 1, 1 - b, (q + 3) % 4)

        i_start(0, 0)
        i_start(1, 1)
        i_start(2, 2)
        g_start(0, 0)
        g_start(1, 1)
        g_start(2, 2)
        g_start(3, 3)
        i_start(3, 3)
        g_wait(0, 0)
        unpack(0, 0)
        g_start(4, 0)
        i_wait(0, 0)
        s_start(0, 0, 0)
        nfour = (NCHUNK - 8) // 4

        def four(t, _):
            i0 = 2 + 4 * t
            for u in range(4):
                step(i0 + u, u % 2, (2 + u) % 4, True, True)
            return 0
        lax.fori_loop(0, nfour, four, 0)
        step(NCHUNK - 6, 0, 2, True, True)
        step(NCHUNK - 5, 1, 3, True, True)
        step(NCHUNK - 4, 0, 0, True, True)
        step(NCHUNK - 3, 1, 1, True, False)
        step(NCHUNK - 2, 0, 2, False, False)
        step(NCHUNK - 1, 1, 3, False, False)
        s_wait(NCHUNK - 2, 0)
        g_wait(NCHUNK - 1, 3)
        unpack(3, 1)
        i_wait(NCHUNK - 1, 3)
        s_start(NCHUNK - 1, 1, 3)
        s_wait(NCHUNK - 1, 1)
        plsc.subcore_barrier()

        pltpu.sync_copy(acc.at[pl.ds(row0, ROWS_PER_TILE)],
                        out_hbm.at[cid, pl.ds(row0, ROWS_PER_TILE)])

    return pl.kernel(
        body,
        out_type=jax.ShapeDtypeStruct((NC, N_PAD, D), jnp.float32),
        compiler_params=pltpu.CompilerParams(use_tc_tiling_on_sc=False,
                                             needs_layout_passes=False),
        mesh=mesh,
        scratch_types=[
            pltpu.VMEM((EPT_PAD,), jnp.int32),
            pltpu.VMEM((4, CHUNK), jnp.int32),
            pltpu.VMEM((4, CHUNK, DW), jnp.int32),
            pltpu.VMEM((2, CHUNK, D), jnp.float32),
            pltpu.VMEM((ZROWS, D), jnp.float32),
            pltpu.VMEM_SHARED((N_PAD, D), jnp.float32),
        ] + [pltpu.SemaphoreType.DMA] * 10,
    )


_propagate = _make_propagate()


_BLK = 2000
_DOT = functools.partial(
    lax.dot_general,
    dimension_numbers=(((1,), (0,)), ((), ())),
    preferred_element_type=jnp.float32,
    precision=lax.Precision.HIGHEST,
)


def _pack_words(y):
    lo = lax.bitcast_convert_type(y[:, :DW].astype(jnp.bfloat16),
                                  jnp.uint16).astype(jnp.int32)
    hi = lax.bitcast_convert_type(y[:, DW:].astype(jnp.bfloat16),
                                  jnp.uint16).astype(jnp.int32)
    return jnp.bitwise_or(lo, lax.shift_left(hi, 16))


def _first_body(x_ref, so_ref, w_ref, bias_ref, o_ref):
    y = _DOT(x_ref[...], w_ref[...]) + bias_ref[...]
    o_ref[...] = _pack_words(y * so_ref[...])


def _mid_body(a_ref, b_ref, si_ref, so_ref, w_ref, bias_ref, o_ref):
    t = (a_ref[...] + b_ref[...]) * si_ref[...]
    t = jnp.maximum(t, 0.0)
    y = _DOT(t, w_ref[...]) + bias_ref[...]
    o_ref[...] = _pack_words(y * so_ref[...])


def _last_body(a_ref, b_ref, si_ref, o_ref):
    o_ref[...] = (a_ref[...] + b_ref[...]) * si_ref[...]


_ROWB = pl.BlockSpec((_BLK, D), lambda i: (i, 0))
_PKB = pl.BlockSpec((_BLK, DW), lambda i: (i, 0))
_COLB = pl.BlockSpec((_BLK, 1), lambda i: (i, 0))
_WB = pl.BlockSpec((D, D), lambda i: (0, 0))
_BB = pl.BlockSpec((1, D), lambda i: (0, 0))
_GRID = (N_ROWS // _BLK,)
_OSHAPE = jax.ShapeDtypeStruct((N_ROWS, D), jnp.float32)
_PSHAPE = jax.ShapeDtypeStruct((N_ROWS, DW), jnp.int32)

_dense_first = pl.pallas_call(
    _first_body, grid=_GRID, out_shape=_PSHAPE,
    in_specs=[_ROWB, _COLB, _WB, _BB], out_specs=_PKB)

_dense_mid = pl.pallas_call(
    _mid_body, grid=_GRID, out_shape=_PSHAPE,
    in_specs=[_ROWB, _ROWB, _COLB, _COLB, _WB, _BB], out_specs=_PKB)

_dense_last = pl.pallas_call(
    _last_body, grid=_GRID, out_shape=_OSHAPE,
    in_specs=[_ROWB, _ROWB, _COLB], out_specs=_ROWB)


def _pad_idx(idx, fill):
    per_tile = idx.reshape(NW, EDGES_PER_TILE)
    return jnp.pad(per_tile, ((0, 0), (0, EPT_PAD - EDGES_PER_TILE)),
                   constant_values=fill)


def kernel(x, edge_index, D_v_beta, D_e_beta_inv, D_e_alpha, D_v_alpha_inv,
           W1v, b1v, W1e, b1e, W2v, b2v, W2e, b2e):
    src = edge_index[0]
    dst = edge_index[1]
    src_g = _pad_idx(src, 0)
    dst_g = _pad_idx(dst, 0)
    src_s = _pad_idx(src, TRASH_ROW).reshape(NW, NCHUNK, CHUNK)
    dst_s = _pad_idx(dst, TRASH_ROW).reshape(NW, NCHUNK, CHUNK)

    dvb = D_v_beta.reshape(N_ROWS, 1)
    debi = D_e_beta_inv.reshape(N_ROWS, 1)
    dea = D_e_alpha.reshape(N_ROWS, 1)
    dvai = D_v_alpha_inv.reshape(N_ROWS, 1)
    b1v_ = b1v.reshape(1, D)
    b1e_ = b1e.reshape(1, D)
    b2v_ = b2v.reshape(1, D)
    b2e_ = b2e.reshape(1, D)

    h1 = _dense_first(x, dvb, W1v, b1v_)
    p = _propagate(h1, src_g, dst_s)
    e2 = _dense_mid(p[0, :N_ROWS], p[1, :N_ROWS], debi, dea, W1e, b1e_)
    q = _propagate(e2, dst_g, src_s)
    h2 = _dense_mid(q[0, :N_ROWS], q[1, :N_ROWS], dvai, dvb, W2v, b2v_)
    p2 = _propagate(h2, src_g, dst_s)
    e2b = _dense_mid(p2[0, :N_ROWS], p2[1, :N_ROWS], debi, dea, W2e, b2e_)
    q2 = _propagate(e2b, dst_g, src_s)
    return _dense_last(q2[0, :N_ROWS], q2[1, :N_ROWS], dvai)
